# baseline (speedup 1.0000x reference)
import jax
import jax.numpy as jnp
from jax import lax
from jax.experimental import pallas as pl
from jax.experimental.pallas import tpu as pltpu

NQ = 6
K = 1024
C = 256
BB = 64
TT = 576
NROWS = BB * TT
R = 256
NH = 2
RH = R // NH
NBLK = NROWS // R


def _vq_kernel(xf_ref, cb_ref, cbsq_ref, cbl_ref, cbm_ref, cbh_ref,
               qo_ref, idx_ref, loss_ref, perp_ref, counts):
    i = pl.program_id(0)
    iih = lax.broadcasted_iota(jnp.int32, (RH, K), 1)
    iif = lax.broadcasted_iota(jnp.int32, (R, K), 1)
    dn = (((1,), (0,)), ((), ()))
    res = [xf_ref[h * RH:(h + 1) * RH, :] for h in range(NH)]
    qac = [jnp.zeros((RH, C), jnp.float32) for _ in range(NH)]
    idx_cols = [[] for _ in range(NH)]
    loss_rows = []
    count_rows = []
    for q in range(NQ):
        cb = cb_ref[q]
        cbsq = cbsq_ref[q]
        idx_h = []
        for h in range(NH):
            r_ = res[h]
            rsq = jnp.sum(r_ * r_, axis=1, keepdims=True)
            cross = lax.dot_general(r_, cb, (((1,), (1,)), ((), ())),
                                    preferred_element_type=jnp.float32)
            d = rsq - 2.0 * cross + cbsq
            m = jnp.min(d, axis=1, keepdims=True)
            idxc = jnp.min(jnp.where(d == m, iih, K), axis=1, keepdims=True)
            idx_h.append(idxc)
            idx_cols[h].append(idxc)
        idxf = jnp.concatenate(idx_h, axis=0)
        ohf = (iif == idxf).astype(jnp.float32)
        oh = ohf.astype(jnp.bfloat16)
        p_lo = lax.dot_general(oh, cbl_ref[q], dn,
                               preferred_element_type=jnp.float32)
        p_mid = lax.dot_general(oh, cbm_ref[q], dn,
                                preferred_element_type=jnp.float32)
        p_hi = lax.dot_general(oh, cbh_ref[q], dn,
                               preferred_element_type=jnp.float32)
        x_df = (p_lo + p_mid) + p_hi
        count_rows.append(jnp.sum(ohf, axis=0, keepdims=True))
        sq_h = []
        for h in range(NH):
            r_ = res[h]
            x_d = x_df[h * RH:(h + 1) * RH, :]
            qz = r_ + (x_d - r_)
            rloss = r_ - x_d
            r_ = r_ - qz
            res[h] = r_
            qac[h] = qac[h] + qz
            sq_h.append(jnp.sum(rloss * rloss))
        loss_rows.append(jnp.full((1, 128), sum(sq_h[1:], sq_h[0]), jnp.float32))

    qo_ref[...] = jnp.concatenate(qac, axis=0)
    idx_ref[...] = jnp.concatenate(
        [jnp.concatenate(cols + [jnp.zeros((RH, 8 - NQ), jnp.int32)], axis=1)
         for cols in idx_cols], axis=0)
    loss_blk = jnp.concatenate(
        loss_rows + [jnp.zeros((8 - NQ, 128), jnp.float32)], axis=0)
    counts_blk = jnp.concatenate(
        count_rows + [jnp.zeros((8 - NQ, K), jnp.float32)], axis=0)

    @pl.when(i == 0)
    def _():
        counts[...] = counts_blk
        loss_ref[...] = loss_blk

    @pl.when(i > 0)
    def _():
        counts[...] += counts_blk
        loss_ref[...] += loss_blk

    @pl.when(i == NBLK - 1)
    def _():
        prob = counts[...] * (1.0 / NROWS)
        plog = prob * jnp.log(prob + 1e-7)
        s = jnp.sum(plog, axis=1, keepdims=True)
        perp_ref[...] = jnp.broadcast_to(jnp.exp(-s), (8, 128))


def kernel(x, codebooks):
    xf = x.transpose(0, 2, 1).reshape(NROWS, C)
    cbsq = jnp.sum(codebooks ** 2, axis=-1).reshape(NQ, 1, K)
    bits = lax.bitcast_convert_type(codebooks, jnp.int32)
    hi_f = lax.bitcast_convert_type(bits & jnp.int32(-65536), jnp.float32)
    rem = codebooks - hi_f
    rbits = lax.bitcast_convert_type(rem, jnp.int32)
    mid_f = lax.bitcast_convert_type(rbits & jnp.int32(-65536), jnp.float32)
    lo_f = rem - mid_f
    cb_hi = hi_f.astype(jnp.bfloat16)
    cb_mid = mid_f.astype(jnp.bfloat16)
    cb_lo = lo_f.astype(jnp.bfloat16)
    qo_flat, idx8, loss8, perp8 = pl.pallas_call(
        _vq_kernel,
        grid=(NBLK,),
        in_specs=[
            pl.BlockSpec((R, C), lambda i: (i, 0)),
            pl.BlockSpec((NQ, K, C), lambda i: (0, 0, 0)),
            pl.BlockSpec((NQ, 1, K), lambda i: (0, 0, 0)),
            pl.BlockSpec((NQ, K, C), lambda i: (0, 0, 0)),
            pl.BlockSpec((NQ, K, C), lambda i: (0, 0, 0)),
            pl.BlockSpec((NQ, K, C), lambda i: (0, 0, 0)),
        ],
        out_specs=[
            pl.BlockSpec((R, C), lambda i: (i, 0)),
            pl.BlockSpec((R, 8), lambda i: (i, 0)),
            pl.BlockSpec((8, 128), lambda i: (0, 0)),
            pl.BlockSpec((8, 128), lambda i: (0, 0)),
        ],
        out_shape=[
            jax.ShapeDtypeStruct((NROWS, C), jnp.float32),
            jax.ShapeDtypeStruct((NROWS, 8), jnp.int32),
            jax.ShapeDtypeStruct((8, 128), jnp.float32),
            jax.ShapeDtypeStruct((8, 128), jnp.float32),
        ],
        scratch_shapes=[pltpu.VMEM((8, K), jnp.float32)],
    )(xf, codebooks, cbsq, cb_lo, cb_mid, cb_hi)
    qo = qo_flat.reshape(BB, TT, C).transpose(0, 2, 1)
    indices = idx8[:, :NQ].reshape(BB, TT, NQ)
    losses = loss8[:NQ, 0] / (NROWS * C)
    perp = perp8[:NQ, 0]
    return qo, indices, losses, perp


# speedup vs baseline: 1.0035x; 1.0035x over previous
import jax
import jax.numpy as jnp
from jax import lax
from jax.experimental import pallas as pl
from jax.experimental.pallas import tpu as pltpu

NQ = 6
K = 1024
C = 256
BB = 64
TT = 576
NROWS = BB * TT
R = 256
NH = 2
RH = R // NH
NBLK = NROWS // R


def _vq_kernel(xf_ref, cb_ref, cbsq_ref, cb3_ref,
               qo_ref, idx_ref, loss_ref, perp_ref, counts):
    i = pl.program_id(0)
    iih = lax.broadcasted_iota(jnp.int32, (RH, K), 1)
    iif = lax.broadcasted_iota(jnp.int32, (R, K), 1)
    dn = (((1,), (0,)), ((), ()))
    res = [xf_ref[h * RH:(h + 1) * RH, :] for h in range(NH)]
    qac = [jnp.zeros((RH, C), jnp.float32) for _ in range(NH)]
    idx_cols = [[] for _ in range(NH)]
    loss_rows = []
    count_rows = []
    for q in range(NQ):
        cb = cb_ref[q]
        cbsq = cbsq_ref[q]
        idx_h = []
        for h in range(NH):
            r_ = res[h]
            rsq = jnp.sum(r_ * r_, axis=1, keepdims=True)
            cross = lax.dot_general(r_, cb, (((1,), (1,)), ((), ())),
                                    preferred_element_type=jnp.float32)
            d = rsq - 2.0 * cross + cbsq
            m = jnp.min(d, axis=1, keepdims=True)
            idxc = jnp.min(jnp.where(d == m, iih, K), axis=1, keepdims=True)
            idx_h.append(idxc)
            idx_cols[h].append(idxc)
        idxf = jnp.concatenate(idx_h, axis=0)
        ohf = (iif == idxf).astype(jnp.float32)
        oh = ohf.astype(jnp.bfloat16)
        oh3 = jnp.concatenate([oh, oh, oh], axis=1)
        x_df = lax.dot_general(oh3, cb3_ref[q], dn,
                               preferred_element_type=jnp.float32)
        count_rows.append(jnp.sum(ohf, axis=0, keepdims=True))
        sq_h = []
        for h in range(NH):
            r_ = res[h]
            x_d = x_df[h * RH:(h + 1) * RH, :]
            qz = r_ + (x_d - r_)
            rloss = r_ - x_d
            r_ = r_ - qz
            res[h] = r_
            qac[h] = qac[h] + qz
            sq_h.append(jnp.sum(rloss * rloss))
        loss_rows.append(jnp.full((1, 128), sum(sq_h[1:], sq_h[0]), jnp.float32))

    qo_ref[...] = jnp.concatenate(qac, axis=0)
    idx_ref[...] = jnp.concatenate(
        [jnp.concatenate(cols + [jnp.zeros((RH, 8 - NQ), jnp.int32)], axis=1)
         for cols in idx_cols], axis=0)
    loss_blk = jnp.concatenate(
        loss_rows + [jnp.zeros((8 - NQ, 128), jnp.float32)], axis=0)
    counts_blk = jnp.concatenate(
        count_rows + [jnp.zeros((8 - NQ, K), jnp.float32)], axis=0)

    @pl.when(i == 0)
    def _():
        counts[...] = counts_blk
        loss_ref[...] = loss_blk

    @pl.when(i > 0)
    def _():
        counts[...] += counts_blk
        loss_ref[...] += loss_blk

    @pl.when(i == NBLK - 1)
    def _():
        prob = counts[...] * (1.0 / NROWS)
        plog = prob * jnp.log(prob + 1e-7)
        s = jnp.sum(plog, axis=1, keepdims=True)
        perp_ref[...] = jnp.broadcast_to(jnp.exp(-s), (8, 128))


def kernel(x, codebooks):
    xf = x.transpose(0, 2, 1).reshape(NROWS, C)
    cbsq = jnp.sum(codebooks ** 2, axis=-1).reshape(NQ, 1, K)
    bits = lax.bitcast_convert_type(codebooks, jnp.int32)
    hi_f = lax.bitcast_convert_type(bits & jnp.int32(-65536), jnp.float32)
    rem = codebooks - hi_f
    rbits = lax.bitcast_convert_type(rem, jnp.int32)
    mid_f = lax.bitcast_convert_type(rbits & jnp.int32(-65536), jnp.float32)
    lo_f = rem - mid_f
    cb_hi = hi_f.astype(jnp.bfloat16)
    cb_mid = mid_f.astype(jnp.bfloat16)
    cb_lo = lo_f.astype(jnp.bfloat16)
    cb3 = jnp.concatenate([cb_lo, cb_mid, cb_hi], axis=1)
    qo_flat, idx8, loss8, perp8 = pl.pallas_call(
        _vq_kernel,
        grid=(NBLK,),
        in_specs=[
            pl.BlockSpec((R, C), lambda i: (i, 0)),
            pl.BlockSpec((NQ, K, C), lambda i: (0, 0, 0)),
            pl.BlockSpec((NQ, 1, K), lambda i: (0, 0, 0)),
            pl.BlockSpec((NQ, 3 * K, C), lambda i: (0, 0, 0)),
        ],
        out_specs=[
            pl.BlockSpec((R, C), lambda i: (i, 0)),
            pl.BlockSpec((R, 8), lambda i: (i, 0)),
            pl.BlockSpec((8, 128), lambda i: (0, 0)),
            pl.BlockSpec((8, 128), lambda i: (0, 0)),
        ],
        out_shape=[
            jax.ShapeDtypeStruct((NROWS, C), jnp.float32),
            jax.ShapeDtypeStruct((NROWS, 8), jnp.int32),
            jax.ShapeDtypeStruct((8, 128), jnp.float32),
            jax.ShapeDtypeStruct((8, 128), jnp.float32),
        ],
        scratch_shapes=[pltpu.VMEM((8, K), jnp.float32)],
    )(xf, codebooks, cbsq, cb3)
    qo = qo_flat.reshape(BB, TT, C).transpose(0, 2, 1)
    indices = idx8[:, :NQ].reshape(BB, TT, NQ)
    losses = loss8[:NQ, 0] / (NROWS * C)
    perp = perp8[:NQ, 0]
    return qo, indices, losses, perp


# speedup vs baseline: 1.3841x; 1.3793x over previous
import jax
import jax.numpy as jnp
from jax import lax
from jax.experimental import pallas as pl
from jax.experimental.pallas import tpu as pltpu

NQ = 6
K = 1024
C = 256
BB = 64
TT = 576
NROWS = BB * TT
R = 256
NH = 2
RH = R // NH
NBLK = NROWS // R


def _vq_kernel(xf_ref, cb_ref, cbsq_ref, cb3_ref,
               qo_ref, idx_ref, loss_ref, perp_ref, counts):
    i = pl.program_id(0)
    iih = lax.broadcasted_iota(jnp.int32, (RH, K), 1)
    iif = lax.broadcasted_iota(jnp.int32, (R, K), 1)
    dn = (((1,), (0,)), ((), ()))
    res = [xf_ref[h * RH:(h + 1) * RH, :] for h in range(NH)]
    qac = [jnp.zeros((RH, C), jnp.float32) for _ in range(NH)]
    idx_cols = [[] for _ in range(NH)]
    loss_rows = []
    count_rows = []
    for q in range(NQ):
        cb = cb_ref[q]
        cbsq = cbsq_ref[q]
        counts_h = []
        sq_h = []
        for h in range(NH):
            r_ = res[h]
            rsq = jnp.sum(r_ * r_, axis=1, keepdims=True)
            cross = lax.dot_general(r_, cb, (((1,), (1,)), ((), ())),
                                    preferred_element_type=jnp.float32)
            d = rsq - 2.0 * cross + cbsq
            m = jnp.min(d, axis=1, keepdims=True)
            idxc = jnp.min(jnp.where(d == m, iih, K), axis=1, keepdims=True)
            ohf = (iih == idxc).astype(jnp.float32)
            oh = ohf.astype(jnp.bfloat16)
            oh3 = jnp.concatenate([oh, oh, oh], axis=1)
            x_d = lax.dot_general(oh3, cb3_ref[q], dn,
                                  preferred_element_type=jnp.float32)
            qz = r_ + (x_d - r_)
            rloss = r_ - x_d
            r_ = r_ - qz
            res[h] = r_
            qac[h] = qac[h] + qz
            counts_h.append(jnp.sum(ohf, axis=0, keepdims=True))
            sq_h.append(jnp.sum(rloss * rloss))
            idx_cols[h].append(idxc)
        count_rows.append(sum(counts_h[1:], counts_h[0]))
        loss_rows.append(jnp.full((1, 128), sum(sq_h[1:], sq_h[0]), jnp.float32))

    qo_ref[...] = jnp.concatenate(qac, axis=0)
    idx_ref[...] = jnp.concatenate(
        [jnp.concatenate(cols + [jnp.zeros((RH, 8 - NQ), jnp.int32)], axis=1)
         for cols in idx_cols], axis=0)
    loss_blk = jnp.concatenate(
        loss_rows + [jnp.zeros((8 - NQ, 128), jnp.float32)], axis=0)
    counts_blk = jnp.concatenate(
        count_rows + [jnp.zeros((8 - NQ, K), jnp.float32)], axis=0)

    @pl.when(i == 0)
    def _():
        counts[...] = counts_blk
        loss_ref[...] = loss_blk

    @pl.when(i > 0)
    def _():
        counts[...] += counts_blk
        loss_ref[...] += loss_blk

    @pl.when(i == NBLK - 1)
    def _():
        prob = counts[...] * (1.0 / NROWS)
        plog = prob * jnp.log(prob + 1e-7)
        s = jnp.sum(plog, axis=1, keepdims=True)
        perp_ref[...] = jnp.broadcast_to(jnp.exp(-s), (8, 128))


def kernel(x, codebooks):
    xf = x.transpose(0, 2, 1).reshape(NROWS, C)
    cbsq = jnp.sum(codebooks ** 2, axis=-1).reshape(NQ, 1, K)
    bits = lax.bitcast_convert_type(codebooks, jnp.int32)
    hi_f = lax.bitcast_convert_type(bits & jnp.int32(-65536), jnp.float32)
    rem = codebooks - hi_f
    rbits = lax.bitcast_convert_type(rem, jnp.int32)
    mid_f = lax.bitcast_convert_type(rbits & jnp.int32(-65536), jnp.float32)
    lo_f = rem - mid_f
    cb_hi = hi_f.astype(jnp.bfloat16)
    cb_mid = mid_f.astype(jnp.bfloat16)
    cb_lo = lo_f.astype(jnp.bfloat16)
    cb3 = jnp.concatenate([cb_lo, cb_mid, cb_hi], axis=1)
    qo_flat, idx8, loss8, perp8 = pl.pallas_call(
        _vq_kernel,
        grid=(NBLK,),
        in_specs=[
            pl.BlockSpec((R, C), lambda i: (i, 0)),
            pl.BlockSpec((NQ, K, C), lambda i: (0, 0, 0)),
            pl.BlockSpec((NQ, 1, K), lambda i: (0, 0, 0)),
            pl.BlockSpec((NQ, 3 * K, C), lambda i: (0, 0, 0)),
        ],
        out_specs=[
            pl.BlockSpec((R, C), lambda i: (i, 0)),
            pl.BlockSpec((R, 8), lambda i: (i, 0)),
            pl.BlockSpec((8, 128), lambda i: (0, 0)),
            pl.BlockSpec((8, 128), lambda i: (0, 0)),
        ],
        out_shape=[
            jax.ShapeDtypeStruct((NROWS, C), jnp.float32),
            jax.ShapeDtypeStruct((NROWS, 8), jnp.int32),
            jax.ShapeDtypeStruct((8, 128), jnp.float32),
            jax.ShapeDtypeStruct((8, 128), jnp.float32),
        ],
        scratch_shapes=[pltpu.VMEM((8, K), jnp.float32)],
    )(xf, codebooks, cbsq, cb3)
    qo = qo_flat.reshape(BB, TT, C).transpose(0, 2, 1)
    indices = idx8[:, :NQ].reshape(BB, TT, NQ)
    losses = loss8[:NQ, 0] / (NROWS * C)
    perp = perp8[:NQ, 0]
    return qo, indices, losses, perp


# speedup vs baseline: 1.9276x; 1.3926x over previous
import jax
import jax.numpy as jnp
from jax import lax
from jax.experimental import pallas as pl
from jax.experimental.pallas import tpu as pltpu

NQ = 6
K = 1024
C = 256
BB = 64
TT = 576
NROWS = BB * TT
R = 512
NH = 2
RH = R // NH
NBLK = NROWS // R


def _vq_kernel(xf_ref, cb_ref, cbsq_ref, cb3_ref,
               qo_ref, idx_ref, loss_ref, perp_ref, counts):
    i = pl.program_id(0)
    iih = lax.broadcasted_iota(jnp.int32, (RH, K), 1)
    iif = lax.broadcasted_iota(jnp.int32, (R, K), 1)
    dn = (((1,), (0,)), ((), ()))
    res = [xf_ref[h * RH:(h + 1) * RH, :] for h in range(NH)]
    qac = [jnp.zeros((RH, C), jnp.float32) for _ in range(NH)]
    idx_cols = [[] for _ in range(NH)]
    loss_rows = []
    count_rows = []
    for q in range(NQ):
        cb = cb_ref[q]
        cbsq = cbsq_ref[q]
        counts_h = []
        sq_h = []
        for h in range(NH):
            r_ = res[h]
            rsq = jnp.sum(r_ * r_, axis=1, keepdims=True)
            cross = lax.dot_general(r_, cb, (((1,), (1,)), ((), ())),
                                    preferred_element_type=jnp.float32)
            d = rsq - 2.0 * cross + cbsq
            m = jnp.min(d, axis=1, keepdims=True)
            idxc = jnp.min(jnp.where(d == m, iih, K), axis=1, keepdims=True)
            ohf = (iih == idxc).astype(jnp.float32)
            oh = ohf.astype(jnp.bfloat16)
            oh3 = jnp.concatenate([oh, oh, oh], axis=1)
            x_d = lax.dot_general(oh3, cb3_ref[q], dn,
                                  preferred_element_type=jnp.float32)
            qz = r_ + (x_d - r_)
            rloss = r_ - x_d
            r_ = r_ - qz
            res[h] = r_
            qac[h] = qac[h] + qz
            counts_h.append(jnp.sum(ohf, axis=0, keepdims=True))
            sq_h.append(jnp.sum(rloss * rloss))
            idx_cols[h].append(idxc)
        count_rows.append(sum(counts_h[1:], counts_h[0]))
        loss_rows.append(jnp.full((1, 128), sum(sq_h[1:], sq_h[0]), jnp.float32))

    qo_ref[...] = jnp.concatenate(qac, axis=0)
    idx_ref[...] = jnp.concatenate(
        [jnp.concatenate(cols + [jnp.zeros((RH, 8 - NQ), jnp.int32)], axis=1)
         for cols in idx_cols], axis=0)
    loss_blk = jnp.concatenate(
        loss_rows + [jnp.zeros((8 - NQ, 128), jnp.float32)], axis=0)
    counts_blk = jnp.concatenate(
        count_rows + [jnp.zeros((8 - NQ, K), jnp.float32)], axis=0)

    @pl.when(i == 0)
    def _():
        counts[...] = counts_blk
        loss_ref[...] = loss_blk

    @pl.when(i > 0)
    def _():
        counts[...] += counts_blk
        loss_ref[...] += loss_blk

    @pl.when(i == NBLK - 1)
    def _():
        prob = counts[...] * (1.0 / NROWS)
        plog = prob * jnp.log(prob + 1e-7)
        s = jnp.sum(plog, axis=1, keepdims=True)
        perp_ref[...] = jnp.broadcast_to(jnp.exp(-s), (8, 128))


def kernel(x, codebooks):
    xf = x.transpose(0, 2, 1).reshape(NROWS, C)
    cbsq = jnp.sum(codebooks ** 2, axis=-1).reshape(NQ, 1, K)
    bits = lax.bitcast_convert_type(codebooks, jnp.int32)
    hi_f = lax.bitcast_convert_type(bits & jnp.int32(-65536), jnp.float32)
    rem = codebooks - hi_f
    rbits = lax.bitcast_convert_type(rem, jnp.int32)
    mid_f = lax.bitcast_convert_type(rbits & jnp.int32(-65536), jnp.float32)
    lo_f = rem - mid_f
    cb_hi = hi_f.astype(jnp.bfloat16)
    cb_mid = mid_f.astype(jnp.bfloat16)
    cb_lo = lo_f.astype(jnp.bfloat16)
    cb3 = jnp.concatenate([cb_lo, cb_mid, cb_hi], axis=1)
    qo_flat, idx8, loss8, perp8 = pl.pallas_call(
        _vq_kernel,
        grid=(NBLK,),
        in_specs=[
            pl.BlockSpec((R, C), lambda i: (i, 0)),
            pl.BlockSpec((NQ, K, C), lambda i: (0, 0, 0)),
            pl.BlockSpec((NQ, 1, K), lambda i: (0, 0, 0)),
            pl.BlockSpec((NQ, 3 * K, C), lambda i: (0, 0, 0)),
        ],
        out_specs=[
            pl.BlockSpec((R, C), lambda i: (i, 0)),
            pl.BlockSpec((R, 8), lambda i: (i, 0)),
            pl.BlockSpec((8, 128), lambda i: (0, 0)),
            pl.BlockSpec((8, 128), lambda i: (0, 0)),
        ],
        out_shape=[
            jax.ShapeDtypeStruct((NROWS, C), jnp.float32),
            jax.ShapeDtypeStruct((NROWS, 8), jnp.int32),
            jax.ShapeDtypeStruct((8, 128), jnp.float32),
            jax.ShapeDtypeStruct((8, 128), jnp.float32),
        ],
        scratch_shapes=[pltpu.VMEM((8, K), jnp.float32)],
    )(xf, codebooks, cbsq, cb3)
    qo = qo_flat.reshape(BB, TT, C).transpose(0, 2, 1)
    indices = idx8[:, :NQ].reshape(BB, TT, NQ)
    losses = loss8[:NQ, 0] / (NROWS * C)
    perp = perp8[:NQ, 0]
    return qo, indices, losses, perp


# speedup vs baseline: 1.9276x; 1.0000x over previous
import jax
import jax.numpy as jnp
from jax import lax
from jax.experimental import pallas as pl
from jax.experimental.pallas import tpu as pltpu

NQ = 6
K = 1024
C = 256
BB = 64
TT = 576
NROWS = BB * TT
R = 1024
NH = 4
RH = R // NH
NBLK = NROWS // R


def _vq_kernel(xf_ref, cb_ref, cbsq_ref, cb3_ref,
               qo_ref, idx_ref, loss_ref, perp_ref, counts):
    i = pl.program_id(0)
    iih = lax.broadcasted_iota(jnp.int32, (RH, K), 1)
    iif = lax.broadcasted_iota(jnp.int32, (R, K), 1)
    dn = (((1,), (0,)), ((), ()))
    res = [xf_ref[h * RH:(h + 1) * RH, :] for h in range(NH)]
    qac = [jnp.zeros((RH, C), jnp.float32) for _ in range(NH)]
    idx_cols = [[] for _ in range(NH)]
    loss_rows = []
    count_rows = []
    for q in range(NQ):
        cb = cb_ref[q]
        cbsq = cbsq_ref[q]
        counts_h = []
        sq_h = []
        for h in range(NH):
            r_ = res[h]
            rsq = jnp.sum(r_ * r_, axis=1, keepdims=True)
            cross = lax.dot_general(r_, cb, (((1,), (1,)), ((), ())),
                                    preferred_element_type=jnp.float32)
            d = rsq - 2.0 * cross + cbsq
            m = jnp.min(d, axis=1, keepdims=True)
            idxc = jnp.min(jnp.where(d == m, iih, K), axis=1, keepdims=True)
            ohf = (iih == idxc).astype(jnp.float32)
            oh = ohf.astype(jnp.bfloat16)
            oh3 = jnp.concatenate([oh, oh, oh], axis=1)
            x_d = lax.dot_general(oh3, cb3_ref[q], dn,
                                  preferred_element_type=jnp.float32)
            qz = r_ + (x_d - r_)
            rloss = r_ - x_d
            r_ = r_ - qz
            res[h] = r_
            qac[h] = qac[h] + qz
            counts_h.append(jnp.sum(ohf, axis=0, keepdims=True))
            sq_h.append(jnp.sum(rloss * rloss))
            idx_cols[h].append(idxc)
        count_rows.append(sum(counts_h[1:], counts_h[0]))
        loss_rows.append(jnp.full((1, 128), sum(sq_h[1:], sq_h[0]), jnp.float32))

    qo_ref[...] = jnp.concatenate(qac, axis=0)
    idx_ref[...] = jnp.concatenate(
        [jnp.concatenate(cols + [jnp.zeros((RH, 8 - NQ), jnp.int32)], axis=1)
         for cols in idx_cols], axis=0)
    loss_blk = jnp.concatenate(
        loss_rows + [jnp.zeros((8 - NQ, 128), jnp.float32)], axis=0)
    counts_blk = jnp.concatenate(
        count_rows + [jnp.zeros((8 - NQ, K), jnp.float32)], axis=0)

    @pl.when(i == 0)
    def _():
        counts[...] = counts_blk
        loss_ref[...] = loss_blk

    @pl.when(i > 0)
    def _():
        counts[...] += counts_blk
        loss_ref[...] += loss_blk

    @pl.when(i == NBLK - 1)
    def _():
        prob = counts[...] * (1.0 / NROWS)
        plog = prob * jnp.log(prob + 1e-7)
        s = jnp.sum(plog, axis=1, keepdims=True)
        perp_ref[...] = jnp.broadcast_to(jnp.exp(-s), (8, 128))


def kernel(x, codebooks):
    xf = x.transpose(0, 2, 1).reshape(NROWS, C)
    cbsq = jnp.sum(codebooks ** 2, axis=-1).reshape(NQ, 1, K)
    bits = lax.bitcast_convert_type(codebooks, jnp.int32)
    hi_f = lax.bitcast_convert_type(bits & jnp.int32(-65536), jnp.float32)
    rem = codebooks - hi_f
    rbits = lax.bitcast_convert_type(rem, jnp.int32)
    mid_f = lax.bitcast_convert_type(rbits & jnp.int32(-65536), jnp.float32)
    lo_f = rem - mid_f
    cb_hi = hi_f.astype(jnp.bfloat16)
    cb_mid = mid_f.astype(jnp.bfloat16)
    cb_lo = lo_f.astype(jnp.bfloat16)
    cb3 = jnp.concatenate([cb_lo, cb_mid, cb_hi], axis=1)
    qo_flat, idx8, loss8, perp8 = pl.pallas_call(
        _vq_kernel,
        grid=(NBLK,),
        in_specs=[
            pl.BlockSpec((R, C), lambda i: (i, 0)),
            pl.BlockSpec((NQ, K, C), lambda i: (0, 0, 0)),
            pl.BlockSpec((NQ, 1, K), lambda i: (0, 0, 0)),
            pl.BlockSpec((NQ, 3 * K, C), lambda i: (0, 0, 0)),
        ],
        out_specs=[
            pl.BlockSpec((R, C), lambda i: (i, 0)),
            pl.BlockSpec((R, 8), lambda i: (i, 0)),
            pl.BlockSpec((8, 128), lambda i: (0, 0)),
            pl.BlockSpec((8, 128), lambda i: (0, 0)),
        ],
        out_shape=[
            jax.ShapeDtypeStruct((NROWS, C), jnp.float32),
            jax.ShapeDtypeStruct((NROWS, 8), jnp.int32),
            jax.ShapeDtypeStruct((8, 128), jnp.float32),
            jax.ShapeDtypeStruct((8, 128), jnp.float32),
        ],
        scratch_shapes=[pltpu.VMEM((8, K), jnp.float32)],
    )(xf, codebooks, cbsq, cb3)
    qo = qo_flat.reshape(BB, TT, C).transpose(0, 2, 1)
    indices = idx8[:, :NQ].reshape(BB, TT, NQ)
    losses = loss8[:NQ, 0] / (NROWS * C)
    perp = perp8[:NQ, 0]
    return qo, indices, losses, perp


# speedup vs baseline: 1.9823x; 1.0284x over previous
import jax
import jax.numpy as jnp
from jax import lax
from jax.experimental import pallas as pl
from jax.experimental.pallas import tpu as pltpu

NQ = 6
K = 1024
C = 256
BB = 64
TT = 576
NROWS = BB * TT
R = 1024
NH = 4
RH = R // NH
NBLK = NROWS // R


def _vq_kernel(xf_ref, cb_ref, cbsq_ref, cb3_ref,
               qo_ref, idx_ref, loss_ref, perp_ref, counts):
    i = pl.program_id(0)
    iih = lax.broadcasted_iota(jnp.int32, (RH, K), 1)
    iih16 = iih.astype(jnp.int16)
    dn = (((1,), (0,)), ((), ()))
    res = [xf_ref[h * RH:(h + 1) * RH, :] for h in range(NH)]
    qac = [jnp.zeros((RH, C), jnp.float32) for _ in range(NH)]
    idx_cols = [[] for _ in range(NH)]
    loss_rows = []
    count_rows = []
    for q in range(NQ):
        cb = cb_ref[q]
        cbsq = cbsq_ref[q]
        counts_h = []
        sq_h = []
        for h in range(NH):
            r_ = res[h]
            rsq = jnp.sum(r_ * r_, axis=1, keepdims=True)
            cross = lax.dot_general(r_, cb, (((1,), (1,)), ((), ())),
                                    preferred_element_type=jnp.float32)
            d = rsq - 2.0 * cross + cbsq
            m = jnp.min(d, axis=1, keepdims=True)
            idxc = jnp.min(jnp.where(d == m, iih, K), axis=1, keepdims=True)
            oh = lax.bitcast_convert_type(
                jnp.where(iih16 == idxc.astype(jnp.int16),
                          jnp.int16(0x3F80), jnp.int16(0)),
                jnp.bfloat16)
            oh3 = jnp.concatenate([oh, oh, oh], axis=1)
            x_d = lax.dot_general(oh3, cb3_ref[q], dn,
                                  preferred_element_type=jnp.float32)
            qz = r_ + (x_d - r_)
            rloss = r_ - x_d
            r_ = r_ - qz
            res[h] = r_
            qac[h] = qac[h] + qz
            counts_h.append(jnp.sum(oh, axis=0, keepdims=True).astype(jnp.float32))
            sq_h.append(jnp.sum(rloss * rloss))
            idx_cols[h].append(idxc)
        count_rows.append(sum(counts_h[1:], counts_h[0]))
        loss_rows.append(jnp.full((1, 128), sum(sq_h[1:], sq_h[0]), jnp.float32))

    qo_ref[...] = jnp.concatenate(qac, axis=0)
    idx_ref[...] = jnp.concatenate(
        [jnp.concatenate(cols + [jnp.zeros((RH, 8 - NQ), jnp.int32)], axis=1)
         for cols in idx_cols], axis=0)
    loss_blk = jnp.concatenate(
        loss_rows + [jnp.zeros((8 - NQ, 128), jnp.float32)], axis=0)
    counts_blk = jnp.concatenate(
        count_rows + [jnp.zeros((8 - NQ, K), jnp.float32)], axis=0)

    @pl.when(i == 0)
    def _():
        counts[...] = counts_blk
        loss_ref[...] = loss_blk

    @pl.when(i > 0)
    def _():
        counts[...] += counts_blk
        loss_ref[...] += loss_blk

    @pl.when(i == NBLK - 1)
    def _():
        prob = counts[...] * (1.0 / NROWS)
        plog = prob * jnp.log(prob + 1e-7)
        s = jnp.sum(plog, axis=1, keepdims=True)
        perp_ref[...] = jnp.broadcast_to(jnp.exp(-s), (8, 128))


def kernel(x, codebooks):
    xf = x.transpose(0, 2, 1).reshape(NROWS, C)
    cbsq = jnp.sum(codebooks ** 2, axis=-1).reshape(NQ, 1, K)
    bits = lax.bitcast_convert_type(codebooks, jnp.int32)
    hi_f = lax.bitcast_convert_type(bits & jnp.int32(-65536), jnp.float32)
    rem = codebooks - hi_f
    rbits = lax.bitcast_convert_type(rem, jnp.int32)
    mid_f = lax.bitcast_convert_type(rbits & jnp.int32(-65536), jnp.float32)
    lo_f = rem - mid_f
    cb_hi = hi_f.astype(jnp.bfloat16)
    cb_mid = mid_f.astype(jnp.bfloat16)
    cb_lo = lo_f.astype(jnp.bfloat16)
    cb3 = jnp.concatenate([cb_lo, cb_mid, cb_hi], axis=1)
    qo_flat, idx8, loss8, perp8 = pl.pallas_call(
        _vq_kernel,
        grid=(NBLK,),
        in_specs=[
            pl.BlockSpec((R, C), lambda i: (i, 0)),
            pl.BlockSpec((NQ, K, C), lambda i: (0, 0, 0)),
            pl.BlockSpec((NQ, 1, K), lambda i: (0, 0, 0)),
            pl.BlockSpec((NQ, 3 * K, C), lambda i: (0, 0, 0)),
        ],
        out_specs=[
            pl.BlockSpec((R, C), lambda i: (i, 0)),
            pl.BlockSpec((R, 8), lambda i: (i, 0)),
            pl.BlockSpec((8, 128), lambda i: (0, 0)),
            pl.BlockSpec((8, 128), lambda i: (0, 0)),
        ],
        out_shape=[
            jax.ShapeDtypeStruct((NROWS, C), jnp.float32),
            jax.ShapeDtypeStruct((NROWS, 8), jnp.int32),
            jax.ShapeDtypeStruct((8, 128), jnp.float32),
            jax.ShapeDtypeStruct((8, 128), jnp.float32),
        ],
        scratch_shapes=[pltpu.VMEM((8, K), jnp.float32)],
    )(xf, codebooks, cbsq, cb3)
    qo = qo_flat.reshape(BB, TT, C).transpose(0, 2, 1)
    indices = idx8[:, :NQ].reshape(BB, TT, NQ)
    losses = loss8[:NQ, 0] / (NROWS * C)
    perp = perp8[:NQ, 0]
    return qo, indices, losses, perp
